# baseline (speedup 1.0000x reference)
import jax
import jax.numpy as jnp
from jax import lax
from jax.experimental import pallas as pl
from jax.experimental.pallas import tpu as pltpu
from jax.experimental.pallas import tpu_sc as plsc

D = 128
L = 16
NC, NS = 2, 16
NW = NC * NS
C = 80
NG = C // L


def _tree_sum(vals):
    vals = list(vals)
    while len(vals) > 1:
        nxt = [a + b for a, b in zip(vals[0::2], vals[1::2])]
        if len(vals) % 2:
            nxt.append(vals[-1])
        vals = nxt
    return vals[0]


def _decoder_body(z_hbm, src_hbm, dst_hbm, out_hbm,
                  sidx0, sidx1, didx0, didx1,
                  srow0, srow1, drow0, drow1,
                  m_v, out0, out1, z_sh,
                  sem_r0, sem_r1, sem_i0, sem_i1, sem_o0, sem_o1):
    num_edges = src_hbm.shape[0]
    ew = num_edges // NW
    nch = ew // C

    wid = lax.axis_index("s") * NC + lax.axis_index("c")
    base_w = wid * ew
    lane = lax.iota(jnp.int32, L)
    scat_base = lane * L

    def issue_idx(ch, si, di, sem):
        base = base_w + ch * C
        pltpu.async_copy(src_hbm.at[pl.ds(base, C)], si, sem)
        pltpu.async_copy(dst_hbm.at[pl.ds(base, C)], di, sem)

    def wait_idx(si, di, sem):
        pltpu.make_async_copy(src_hbm.at[pl.ds(0, C)], si, sem).wait()
        pltpu.make_async_copy(dst_hbm.at[pl.ds(0, C)], di, sem).wait()

    def issue_gather(si, di, sr, dr, sem):
        pltpu.async_copy(z_sh.at[si], sr, sem)
        pltpu.async_copy(z_sh.at[di], dr, sem)

    def wait_gather(si, di, sr, dr, sem):
        pltpu.make_async_copy(z_sh.at[si], sr, sem).wait()
        pltpu.make_async_copy(z_sh.at[di], dr, sem).wait()

    def issue_out(ch, ob, sem):
        pltpu.async_copy(ob, out_hbm.at[pl.ds(base_w + ch * C, C)], sem)

    def wait_out(ob, sem):
        pltpu.make_async_copy(ob, out_hbm.at[pl.ds(0, C)], sem).wait()

    def compute(sr, dr, ob):
        def group_body(g, gcarry):
            rowbase = g * L
            for e in range(0, L, 2):
                r0 = rowbase + e
                r1 = r0 + 1
                a0 = sr[r0, pl.ds(0, L)] * dr[r0, pl.ds(0, L)]
                a1 = sr[r1, pl.ds(0, L)] * dr[r1, pl.ds(0, L)]
                for k in range(1, D // L):
                    a0 = a0 + sr[r0, pl.ds(k * L, L)] * dr[r0, pl.ds(k * L, L)]
                    a1 = a1 + sr[r1, pl.ds(k * L, L)] * dr[r1, pl.ds(k * L, L)]
                m_v[pl.ds(e * L, L)] = a0
                m_v[pl.ds((e + 1) * L, L)] = a1
            tot = _tree_sum([plsc.load_gather(m_v, [scat_base + l])
                             for l in range(L)])
            ob[pl.ds(rowbase, L)] = 1.0 / (1.0 + jnp.exp(-tot))
            return gcarry

        lax.fori_loop(0, NG, group_body, 0)

    @pl.when(lax.axis_index("s") == 0)
    def _():
        pltpu.sync_copy(z_hbm, z_sh)

    plsc.subcore_barrier()

    issue_out(nch - 1, out1, sem_o1)
    pltpu.sync_copy(src_hbm.at[pl.ds(base_w, C)], sidx0)
    pltpu.sync_copy(dst_hbm.at[pl.ds(base_w, C)], didx0)
    issue_gather(sidx0, didx0, srow0, drow0, sem_r0)
    issue_idx(1, sidx1, didx1, sem_i1)
    wait_idx(sidx1, didx1, sem_i1)
    issue_gather(sidx1, didx1, srow1, drow1, sem_r1)
    wait_gather(sidx0, didx0, srow0, drow0, sem_r0)
    issue_idx(2, sidx0, didx0, sem_i0)
    compute(srow0, drow0, out0)
    issue_out(0, out0, sem_o0)

    def pair_body(i, carry):
        c1 = 2 * i + 1
        c2 = 2 * i + 2
        wait_idx(sidx0, didx0, sem_i0)
        issue_gather(sidx0, didx0, srow0, drow0, sem_r0)
        wait_gather(sidx1, didx1, srow1, drow1, sem_r1)
        issue_idx(c1 + 2, sidx1, didx1, sem_i1)
        wait_out(out1, sem_o1)
        compute(srow1, drow1, out1)
        issue_out(c1, out1, sem_o1)
        wait_idx(sidx1, didx1, sem_i1)
        issue_gather(sidx1, didx1, srow1, drow1, sem_r1)
        wait_gather(sidx0, didx0, srow0, drow0, sem_r0)
        issue_idx(c2 + 2, sidx0, didx0, sem_i0)
        wait_out(out0, sem_o0)
        compute(srow0, drow0, out0)
        issue_out(c2, out0, sem_o0)
        return carry

    lax.fori_loop(0, (nch - 3) // 2, pair_body, 0)

    wait_idx(sidx0, didx0, sem_i0)
    issue_gather(sidx0, didx0, srow0, drow0, sem_r0)
    wait_gather(sidx1, didx1, srow1, drow1, sem_r1)
    wait_out(out1, sem_o1)
    compute(srow1, drow1, out1)
    issue_out(nch - 2, out1, sem_o1)
    wait_gather(sidx0, didx0, srow0, drow0, sem_r0)
    wait_out(out0, sem_o0)
    compute(srow0, drow0, out0)
    issue_out(nch - 1, out0, sem_o0)
    wait_out(out1, sem_o1)
    wait_out(out0, sem_o0)


def kernel(z, edge_index):
    num_edges = edge_index.shape[1]
    ei = edge_index.astype(jnp.int32)
    src, dst = ei[0], ei[1]
    mesh = plsc.VectorSubcoreMesh(core_axis_name="c", subcore_axis_name="s",
                                  num_cores=NC, num_subcores=NS)
    k = pl.kernel(
        _decoder_body,
        out_type=jax.ShapeDtypeStruct((num_edges,), jnp.float32),
        mesh=mesh,
        compiler_params=pltpu.CompilerParams(needs_layout_passes=False),
        scratch_types=[
            pltpu.VMEM((C,), jnp.int32),
            pltpu.VMEM((C,), jnp.int32),
            pltpu.VMEM((C,), jnp.int32),
            pltpu.VMEM((C,), jnp.int32),
            pltpu.VMEM((C, D), jnp.float32),
            pltpu.VMEM((C, D), jnp.float32),
            pltpu.VMEM((C, D), jnp.float32),
            pltpu.VMEM((C, D), jnp.float32),
            pltpu.VMEM((L * L,), jnp.float32),
            pltpu.VMEM((C,), jnp.float32),
            pltpu.VMEM((C,), jnp.float32),
            pltpu.VMEM_SHARED(z.shape, jnp.float32),
            pltpu.SemaphoreType.DMA,
            pltpu.SemaphoreType.DMA,
            pltpu.SemaphoreType.DMA,
            pltpu.SemaphoreType.DMA,
            pltpu.SemaphoreType.DMA,
            pltpu.SemaphoreType.DMA,
        ],
    )
    return k(z, src, dst)


# speedup vs baseline: 1.1540x; 1.1540x over previous
import jax
import jax.numpy as jnp
from jax import lax
from jax.experimental import pallas as pl
from jax.experimental.pallas import tpu as pltpu
from jax.experimental.pallas import tpu_sc as plsc

D = 128
L = 16
NC, NS = 2, 16
NW = NC * NS
C = 80
NG = C // L


def _tree_sum(vals):
    vals = list(vals)
    while len(vals) > 1:
        nxt = [a + b for a, b in zip(vals[0::2], vals[1::2])]
        if len(vals) % 2:
            nxt.append(vals[-1])
        vals = nxt
    return vals[0]


def _decoder_body(z_hbm, src_hbm, dst_hbm, out_hbm,
                  sidx0, sidx1, didx0, didx1,
                  srow0, srow1, drow0, drow1,
                  m_v, out0, out1, z_sh,
                  sem_r0, sem_r1, sem_i0, sem_i1, sem_o0, sem_o1):
    num_edges = src_hbm.shape[0]
    ew = num_edges // NW
    nch = ew // C

    wid = lax.axis_index("s") * NC + lax.axis_index("c")
    base_w = wid * ew
    lane = lax.iota(jnp.int32, L)
    scat_base = lane * L

    def issue_idx(ch, si, di, sem):
        base = base_w + ch * C
        pltpu.async_copy(src_hbm.at[pl.ds(base, C)], si, sem)
        pltpu.async_copy(dst_hbm.at[pl.ds(base, C)], di, sem)

    def wait_idx(si, di, sem):
        pltpu.make_async_copy(src_hbm.at[pl.ds(0, C)], si, sem).wait()
        pltpu.make_async_copy(dst_hbm.at[pl.ds(0, C)], di, sem).wait()

    def issue_gather(si, di, sr, dr, sem):
        pltpu.async_copy(z_sh.at[si], sr, sem)
        pltpu.async_copy(z_sh.at[di], dr, sem)

    def wait_gather(si, di, sr, dr, sem):
        pltpu.make_async_copy(z_sh.at[si], sr, sem).wait()
        pltpu.make_async_copy(z_sh.at[di], dr, sem).wait()

    def issue_out(ch, ob, sem):
        pltpu.async_copy(ob, out_hbm.at[pl.ds(base_w + ch * C, C)], sem)

    def wait_out(ob, sem):
        pltpu.make_async_copy(ob, out_hbm.at[pl.ds(0, C)], sem).wait()

    def compute(sr, dr, ob):
        def group_body(g, gcarry):
            rowbase = g * L
            for e in range(0, L, 4):
                rows = [rowbase + e + j for j in range(4)]
                accs = [sr[r, pl.ds(0, L)] * dr[r, pl.ds(0, L)] for r in rows]
                for k in range(1, D // L):
                    for j, r in enumerate(rows):
                        accs[j] = accs[j] + (sr[r, pl.ds(k * L, L)]
                                             * dr[r, pl.ds(k * L, L)])
                for j in range(4):
                    plsc.store_scatter(m_v, [scat_base + (e + j)], accs[j])
            tot = _tree_sum([m_v[pl.ds(l * L, L)] for l in range(L)])
            ob[pl.ds(rowbase, L)] = 1.0 / (1.0 + jnp.exp(-tot))
            return gcarry

        lax.fori_loop(0, NG, group_body, 0)

    @pl.when(lax.axis_index("s") == 0)
    def _():
        pltpu.sync_copy(z_hbm, z_sh)

    plsc.subcore_barrier()

    issue_out(nch - 1, out1, sem_o1)
    pltpu.sync_copy(src_hbm.at[pl.ds(base_w, C)], sidx0)
    pltpu.sync_copy(dst_hbm.at[pl.ds(base_w, C)], didx0)
    issue_gather(sidx0, didx0, srow0, drow0, sem_r0)
    issue_idx(1, sidx1, didx1, sem_i1)
    wait_idx(sidx1, didx1, sem_i1)
    issue_gather(sidx1, didx1, srow1, drow1, sem_r1)
    wait_gather(sidx0, didx0, srow0, drow0, sem_r0)
    issue_idx(2, sidx0, didx0, sem_i0)
    compute(srow0, drow0, out0)
    issue_out(0, out0, sem_o0)

    def pair_body(i, carry):
        c1 = 2 * i + 1
        c2 = 2 * i + 2
        wait_idx(sidx0, didx0, sem_i0)
        issue_gather(sidx0, didx0, srow0, drow0, sem_r0)
        wait_gather(sidx1, didx1, srow1, drow1, sem_r1)
        issue_idx(c1 + 2, sidx1, didx1, sem_i1)
        wait_out(out1, sem_o1)
        compute(srow1, drow1, out1)
        issue_out(c1, out1, sem_o1)
        wait_idx(sidx1, didx1, sem_i1)
        issue_gather(sidx1, didx1, srow1, drow1, sem_r1)
        wait_gather(sidx0, didx0, srow0, drow0, sem_r0)
        issue_idx(c2 + 2, sidx0, didx0, sem_i0)
        wait_out(out0, sem_o0)
        compute(srow0, drow0, out0)
        issue_out(c2, out0, sem_o0)
        return carry

    lax.fori_loop(0, (nch - 3) // 2, pair_body, 0)

    wait_idx(sidx0, didx0, sem_i0)
    issue_gather(sidx0, didx0, srow0, drow0, sem_r0)
    wait_gather(sidx1, didx1, srow1, drow1, sem_r1)
    wait_out(out1, sem_o1)
    compute(srow1, drow1, out1)
    issue_out(nch - 2, out1, sem_o1)
    wait_gather(sidx0, didx0, srow0, drow0, sem_r0)
    wait_out(out0, sem_o0)
    compute(srow0, drow0, out0)
    issue_out(nch - 1, out0, sem_o0)
    wait_out(out1, sem_o1)
    wait_out(out0, sem_o0)


def kernel(z, edge_index):
    num_edges = edge_index.shape[1]
    ei = edge_index.astype(jnp.int32)
    src, dst = ei[0], ei[1]
    mesh = plsc.VectorSubcoreMesh(core_axis_name="c", subcore_axis_name="s",
                                  num_cores=NC, num_subcores=NS)
    k = pl.kernel(
        _decoder_body,
        out_type=jax.ShapeDtypeStruct((num_edges,), jnp.float32),
        mesh=mesh,
        compiler_params=pltpu.CompilerParams(needs_layout_passes=False),
        scratch_types=[
            pltpu.VMEM((C,), jnp.int32),
            pltpu.VMEM((C,), jnp.int32),
            pltpu.VMEM((C,), jnp.int32),
            pltpu.VMEM((C,), jnp.int32),
            pltpu.VMEM((C, D), jnp.float32),
            pltpu.VMEM((C, D), jnp.float32),
            pltpu.VMEM((C, D), jnp.float32),
            pltpu.VMEM((C, D), jnp.float32),
            pltpu.VMEM((L * L,), jnp.float32),
            pltpu.VMEM((C,), jnp.float32),
            pltpu.VMEM((C,), jnp.float32),
            pltpu.VMEM_SHARED(z.shape, jnp.float32),
            pltpu.SemaphoreType.DMA,
            pltpu.SemaphoreType.DMA,
            pltpu.SemaphoreType.DMA,
            pltpu.SemaphoreType.DMA,
            pltpu.SemaphoreType.DMA,
            pltpu.SemaphoreType.DMA,
        ],
    )
    return k(z, src, dst)
